# Initial kernel scaffold; baseline (speedup 1.0000x reference)
import jax
import jax.numpy as jnp
from jax.experimental import pallas as pl


def kernel(embeddings):
    raise NotImplementedError("write your pallas kernel here")


# speedup vs baseline: 22.5665x; 22.5665x over previous
import functools

import jax
import jax.numpy as jnp
from jax import lax
from jax.experimental import pallas as pl
from jax.experimental.pallas import tpu as pltpu
from jax.experimental.pallas import tpu_sc as plsc

N = 4096
D = 64
K = 5
ROWS = 512
NBLK = N // ROWS
NWORKERS = 32
RPW = N // NWORKERS
GPW = RPW // 16


def _topk_body(e_blk_ref, e_all_ref, idx_ref, wd2_ref):
    e_blk = e_blk_ref[...]
    e_all = e_all_ref[...]
    sq_all = jnp.sum(e_all * e_all, axis=1)
    sq_blk = jnp.sum(e_blk * e_blk, axis=1)
    g = lax.dot_general(e_blk, e_all, (((1,), (1,)), ((), ())),
                        preferred_element_type=jnp.float32,
                        precision=lax.Precision.HIGHEST)
    d2 = jnp.maximum(sq_blk[:, None] + sq_all[None, :] - 2.0 * g, 0.0)
    cols = lax.broadcasted_iota(jnp.int32, (ROWS, N), 1)
    work = d2
    idx_cols = []
    wd2_cols = []
    for t in range(K + 1):
        v = jnp.min(work, axis=1)
        j = jnp.argmin(work, axis=1).astype(jnp.int32)
        if t >= 1:
            idx_cols.append(j[:, None])
            wd2_cols.append((jnp.exp(-0.5 * v) * v)[:, None])
        if t < K:
            work = jnp.where(cols == j[:, None], jnp.float32(jnp.inf), work)
    idx_ref[...] = jnp.concatenate(
        idx_cols + [jnp.zeros((ROWS, 8 - K), jnp.int32)], axis=1)
    wd2_ref[...] = jnp.concatenate(
        wd2_cols + [jnp.zeros((ROWS, 8 - K), jnp.float32)], axis=1)


def _knn_topk(embeddings, interpret=False):
    return pl.pallas_call(
        _topk_body,
        grid=(NBLK,),
        in_specs=[
            pl.BlockSpec((ROWS, D), lambda b: (b, 0)),
            pl.BlockSpec((N, D), lambda b: (0, 0)),
        ],
        out_specs=[
            pl.BlockSpec((ROWS, 8), lambda b: (b, 0)),
            pl.BlockSpec((ROWS, 8), lambda b: (b, 0)),
        ],
        out_shape=[
            jax.ShapeDtypeStruct((N, 8), jnp.int32),
            jax.ShapeDtypeStruct((N, 8), jnp.float32),
        ],
        interpret=interpret,
    )(embeddings, embeddings)


def _edge_body(idx_hbm, wd2_hbm, out_hbm, idx_v, wd2_v, acc_v):
    wid = lax.axis_index("s") * 2 + lax.axis_index("c")
    base = wid * RPW
    pltpu.sync_copy(idx_hbm, idx_v)
    pltpu.sync_copy(wd2_hbm.at[pl.ds(base * 8, RPW * 8)], wd2_v)
    lanes = lax.iota(jnp.int32, 16)

    def group(gi, acc):
        i_loc = gi * 16 + lanes
        i_glob = base + i_loc
        total = acc
        for m in range(K):
            jv = plsc.load_gather(idx_v, [i_glob * 8 + m])
            wv = plsc.load_gather(wd2_v, [i_loc * 8 + m])
            mut = jnp.zeros((16,), jnp.bool_)
            for l in range(K):
                nb = plsc.load_gather(idx_v, [jv * 8 + l])
                mut = jnp.logical_or(mut, nb == i_glob)
            total = total + wv * jnp.where(mut, jnp.float32(0.5),
                                           jnp.float32(1.0))
        return total

    acc = lax.fori_loop(0, GPW, group, jnp.zeros((16,), jnp.float32))
    acc_v[...] = acc
    pltpu.sync_copy(acc_v, out_hbm.at[pl.ds(wid * 16, 16)])


_edge_fix = functools.partial(
    pl.kernel,
    out_type=jax.ShapeDtypeStruct((NWORKERS * 16,), jnp.float32),
    mesh=plsc.VectorSubcoreMesh(core_axis_name="c", subcore_axis_name="s"),
    compiler_params=pltpu.CompilerParams(needs_layout_passes=False),
    scratch_types=[
        pltpu.VMEM((N * 8,), jnp.int32),
        pltpu.VMEM((RPW * 8,), jnp.float32),
        pltpu.VMEM((16,), jnp.float32),
    ],
)(_edge_body)


def kernel(embeddings):
    idx, wd2 = _knn_topk(embeddings)
    partials = _edge_fix(idx.reshape(N * 8), wd2.reshape(N * 8))
    return jnp.sum(partials) / jnp.float32(N * N)


# speedup vs baseline: 28.3161x; 1.2548x over previous
import functools

import jax
import jax.numpy as jnp
from jax import lax
from jax.experimental import pallas as pl
from jax.experimental.pallas import tpu as pltpu
from jax.experimental.pallas import tpu_sc as plsc

N = 4096
D = 64
K = 5
ROWS = 512
NBLK = N // ROWS
NWORKERS = 32
RPW = N // NWORKERS
GPW = RPW // 16


def _topk_body(e_blk_ref, e_all_ref, idx_ref, wd2_ref):
    e_blk = e_blk_ref[...]
    e_all = e_all_ref[...]
    sq_all = jnp.sum(e_all * e_all, axis=1)
    sq_blk = jnp.sum(e_blk * e_blk, axis=1)
    g = lax.dot_general(e_blk, e_all, (((1,), (1,)), ((), ())),
                        preferred_element_type=jnp.float32,
                        precision=lax.Precision.HIGHEST)
    d2 = jnp.maximum(sq_blk[:, None] + sq_all[None, :] - 2.0 * g, 0.0)
    cols = lax.broadcasted_iota(jnp.int32, (ROWS, N), 1)
    keys = (lax.bitcast_convert_type(d2, jnp.int32) & jnp.int32(~0xFFF)) | cols
    work = keys
    idx_cols = []
    wd2_cols = []
    for t in range(K + 1):
        m = jnp.min(work, axis=1)
        if t >= 1:
            v = lax.bitcast_convert_type(m & jnp.int32(~0xFFF), jnp.float32)
            idx_cols.append((m & jnp.int32(0xFFF))[:, None])
            wd2_cols.append((jnp.exp(-0.5 * v) * v)[:, None])
        if t < K:
            work = jnp.where(work == m[:, None], jnp.int32(0x7FFFFFFF), work)
    idx_ref[...] = jnp.concatenate(
        idx_cols + [jnp.zeros((ROWS, 8 - K), jnp.int32)], axis=1)
    wd2_ref[...] = jnp.concatenate(
        wd2_cols + [jnp.zeros((ROWS, 8 - K), jnp.float32)], axis=1)


def _knn_topk(embeddings, interpret=False):
    return pl.pallas_call(
        _topk_body,
        grid=(NBLK,),
        in_specs=[
            pl.BlockSpec((ROWS, D), lambda b: (b, 0)),
            pl.BlockSpec((N, D), lambda b: (0, 0)),
        ],
        out_specs=[
            pl.BlockSpec((ROWS, 8), lambda b: (b, 0)),
            pl.BlockSpec((ROWS, 8), lambda b: (b, 0)),
        ],
        out_shape=[
            jax.ShapeDtypeStruct((N, 8), jnp.int32),
            jax.ShapeDtypeStruct((N, 8), jnp.float32),
        ],
        interpret=interpret,
    )(embeddings, embeddings)


def _edge_body(idx_hbm, wd2_hbm, out_hbm, idx_v, wd2_v, acc_v):
    wid = lax.axis_index("s") * 2 + lax.axis_index("c")
    base = wid * RPW
    pltpu.sync_copy(idx_hbm, idx_v)
    pltpu.sync_copy(wd2_hbm.at[pl.ds(base * 8, RPW * 8)], wd2_v)
    lanes = lax.iota(jnp.int32, 16)

    def group(gi, acc):
        i_loc = gi * 16 + lanes
        i_glob = base + i_loc
        total = acc
        for m in range(K):
            jv = plsc.load_gather(idx_v, [i_glob * 8 + m])
            wv = plsc.load_gather(wd2_v, [i_loc * 8 + m])
            mut = jnp.zeros((16,), jnp.bool_)
            for l in range(K):
                nb = plsc.load_gather(idx_v, [jv * 8 + l])
                mut = jnp.logical_or(mut, nb == i_glob)
            total = total + wv * jnp.where(mut, jnp.float32(0.5),
                                           jnp.float32(1.0))
        return total

    acc = lax.fori_loop(0, GPW, group, jnp.zeros((16,), jnp.float32))
    acc_v[...] = acc
    pltpu.sync_copy(acc_v, out_hbm.at[pl.ds(wid * 16, 16)])


_edge_fix = functools.partial(
    pl.kernel,
    out_type=jax.ShapeDtypeStruct((NWORKERS * 16,), jnp.float32),
    mesh=plsc.VectorSubcoreMesh(core_axis_name="c", subcore_axis_name="s"),
    compiler_params=pltpu.CompilerParams(needs_layout_passes=False),
    scratch_types=[
        pltpu.VMEM((N * 8,), jnp.int32),
        pltpu.VMEM((RPW * 8,), jnp.float32),
        pltpu.VMEM((16,), jnp.float32),
    ],
)(_edge_body)


def kernel(embeddings):
    idx, wd2 = _knn_topk(embeddings)
    partials = _edge_fix(idx.reshape(N * 8), wd2.reshape(N * 8))
    return jnp.sum(partials) / jnp.float32(N * N)


# speedup vs baseline: 29.2699x; 1.0337x over previous
import functools

import jax
import jax.numpy as jnp
from jax import lax
from jax.experimental import pallas as pl
from jax.experimental.pallas import tpu as pltpu
from jax.experimental.pallas import tpu_sc as plsc

N = 4096
D = 64
K = 5
ROWS = 512
NBLK = N // ROWS
NWORKERS = 32
RPW = N // NWORKERS
GPW = RPW // 16


def _topk_body(e_blk_ref, e_all_ref, idx_ref, wd2_ref):
    e_blk = e_blk_ref[...]
    e_all = e_all_ref[...]
    sq_all = jnp.sum(e_all * e_all, axis=1)
    sq_blk = jnp.sum(e_blk * e_blk, axis=1)
    g = lax.dot_general(e_blk, e_all, (((1,), (1,)), ((), ())),
                        preferred_element_type=jnp.float32,
                        precision=lax.Precision.HIGHEST)
    d2 = sq_blk[:, None] + sq_all[None, :] - 2.0 * g
    cols = lax.broadcasted_iota(jnp.int32, (ROWS, N), 1)
    keys = (lax.bitcast_convert_type(d2, jnp.int32) & jnp.int32(~0xFFF)) | cols
    idx_cols = []
    wd2_cols = []
    m = jnp.min(keys, axis=1)
    for t in range(K):
        m = jnp.min(jnp.where(keys > m[:, None], keys, jnp.int32(0x7FFFFFFF)),
                    axis=1)
        v = lax.bitcast_convert_type(m & jnp.int32(~0xFFF), jnp.float32)
        idx_cols.append((m & jnp.int32(0xFFF))[:, None])
        wd2_cols.append((jnp.exp(-0.5 * v) * v)[:, None])
    idx_ref[...] = jnp.concatenate(
        idx_cols + [jnp.zeros((ROWS, 8 - K), jnp.int32)], axis=1)
    wd2_ref[...] = jnp.concatenate(
        wd2_cols + [jnp.zeros((ROWS, 8 - K), jnp.float32)], axis=1)


def _knn_topk(embeddings, interpret=False):
    return pl.pallas_call(
        _topk_body,
        grid=(NBLK,),
        in_specs=[
            pl.BlockSpec((ROWS, D), lambda b: (b, 0)),
            pl.BlockSpec((N, D), lambda b: (0, 0)),
        ],
        out_specs=[
            pl.BlockSpec((ROWS, 8), lambda b: (b, 0)),
            pl.BlockSpec((ROWS, 8), lambda b: (b, 0)),
        ],
        out_shape=[
            jax.ShapeDtypeStruct((N, 8), jnp.int32),
            jax.ShapeDtypeStruct((N, 8), jnp.float32),
        ],
        interpret=interpret,
    )(embeddings, embeddings)


def _edge_body(idx_hbm, wd2_hbm, out_hbm, idx_v, wd2_v, acc_v):
    wid = lax.axis_index("s") * 2 + lax.axis_index("c")
    base = wid * RPW
    pltpu.sync_copy(idx_hbm, idx_v)
    pltpu.sync_copy(wd2_hbm.at[pl.ds(base * 8, RPW * 8)], wd2_v)
    lanes = lax.iota(jnp.int32, 16)

    def group(gi, acc):
        i_loc = gi * 16 + lanes
        i_glob = base + i_loc
        total = acc
        for m in range(K):
            jv = plsc.load_gather(idx_v, [i_glob * 8 + m])
            wv = plsc.load_gather(wd2_v, [i_loc * 8 + m])
            mut = jnp.zeros((16,), jnp.bool_)
            for l in range(K):
                nb = plsc.load_gather(idx_v, [jv * 8 + l])
                mut = jnp.logical_or(mut, nb == i_glob)
            total = total + wv * jnp.where(mut, jnp.float32(0.5),
                                           jnp.float32(1.0))
        return total

    acc = lax.fori_loop(0, GPW, group, jnp.zeros((16,), jnp.float32))
    acc_v[...] = acc
    pltpu.sync_copy(acc_v, out_hbm.at[pl.ds(wid * 16, 16)])


_edge_fix = functools.partial(
    pl.kernel,
    out_type=jax.ShapeDtypeStruct((NWORKERS * 16,), jnp.float32),
    mesh=plsc.VectorSubcoreMesh(core_axis_name="c", subcore_axis_name="s"),
    compiler_params=pltpu.CompilerParams(needs_layout_passes=False),
    scratch_types=[
        pltpu.VMEM((N * 8,), jnp.int32),
        pltpu.VMEM((RPW * 8,), jnp.float32),
        pltpu.VMEM((16,), jnp.float32),
    ],
)(_edge_body)


def kernel(embeddings):
    idx, wd2 = _knn_topk(embeddings)
    partials = _edge_fix(idx.reshape(N * 8), wd2.reshape(N * 8))
    return jnp.sum(partials) / jnp.float32(N * N)
